# baseline (speedup 1.0000x reference)
import functools

import jax
import jax.numpy as jnp
from jax import lax
from jax.experimental import pallas as pl
from jax.experimental.pallas import tpu as pltpu
from jax.experimental.pallas import tpu_sc as plsc

B = 16384
D = 32
V = 1000000
NC, NS = 2, 16
NW = NC * NS
BPW = B // NW
LANE = 128
LAST = (V // LANE) * LANE
TAIL = V - LAST
DEPTH = 6
CHK = 128
NCHK = BPW // CHK
ROWW = 128


def _sc_gather_body(ut_g, ut_m, it_g, it_m, aug, aum, aig, aim,
                    us_hbm, up_hbm, is_hbm, ip_hbm,
                    urows_hbm, irows_hbm,
                    sv, pv, rs, rows,
                    b00, b01, b10, b11, b20, b21, b30, b31,
                    b40, b41, b50, b51,
                    sem0, sem1, sem2, sem3, sem4, sem5, ssem):
    wid = lax.axis_index("s") * NC + lax.axis_index("c")
    base = wid * BPW
    phs = ((b00, b01), (b10, b11), (b20, b21), (b30, b31),
           (b40, b41), (b50, b51))
    sems = (sem0, sem1, sem2, sem3, sem4, sem5)
    i16 = lax.iota(jnp.int32, 16)

    def fetch(tab, aux, off, blk, sem, start):
        @pl.when(off < LAST)
        def _full():
            o = pl.multiple_of(off, LANE)
            cp = pltpu.make_async_copy(tab.at[:, pl.ds(o, LANE)], blk, sem)
            cp.start() if start else cp.wait()

        @pl.when(off >= LAST)
        def _tail():
            cp = pltpu.make_async_copy(aux, blk, sem)
            cp.start() if start else cp.wait()

    def one_pass(tab_a, tab_b, aux_a, aux_b, s_hbm, p_hbm, out_hbm):
        pltpu.sync_copy(s_hbm.at[pl.ds(base, BPW)], sv.at[pl.ds(8, BPW)])
        pltpu.sync_copy(p_hbm.at[pl.ds(wid * NCHK, NCHK)], pv)

        cur0 = sv[pl.ds(8, 16)]
        prv0 = sv[pl.ds(7, 16)]
        m0 = ((cur0 >> 7) != (prv0 >> 7)) | (i16 == 0)
        plsc.store_compressed(rs.at[pl.ds(0, 16)], i16, mask=m0)
        n0 = plsc.all_reduce_population_count(m0)[0]

        def chunk(o, off):
            o16 = o * 16
            cur = sv[pl.ds(o16 + 8, 16)]
            prv = sv[pl.ds(o16 + 7, 16)]
            m = (cur >> 7) != (prv >> 7)
            plsc.store_compressed(rs.at[pl.ds(off, 16)], i16 + o16, mask=m)
            return off + plsc.all_reduce_population_count(m)[0]

        n_run = lax.fori_loop(1, BPW // 16, chunk, n0)
        plsc.store_compressed(rs.at[pl.ds(n_run, 16)],
                              jnp.full((16,), BPW, jnp.int32), mask=i16 == 0)

        def fpair(f, blks, sem, start):
            st = rs[pl.ds(f, 16)][0]
            u = sv[pl.ds(st + 8, 16)][0]
            off = u & ~(LANE - 1)
            fetch(tab_a, aux_a, off, blks[0], sem, start)
            fetch(tab_b, aux_b, off, blks[1], sem, start)

        for p in range(DEPTH):
            @pl.when(p < n_run)
            def _pro():
                fpair(p, phs[p], sems[p], True)

        def outer(t, _):
            f0 = t * DEPTH
            for p in range(DEPTH):
                f = f0 + p

                @pl.when(f < n_run)
                def _do():
                    fpair(f, phs[p], sems[p], False)
                    rsv = rs[pl.ds(f, 16)]

                    def ex(j, _c):
                        u = sv[pl.ds(j + 8, 16)][0]
                        l = jnp.full((16,), u & (LANE - 1), jnp.int32)
                        rows[j, pl.ds(0, 16)] = plsc.load_gather(
                            phs[p][0], [i16, l])
                        rows[j, pl.ds(16, 16)] = plsc.load_gather(
                            phs[p][0], [i16 + 16, l])
                        rows[j, pl.ds(32, 16)] = plsc.load_gather(
                            phs[p][1], [i16, l])
                        rows[j, pl.ds(48, 16)] = plsc.load_gather(
                            phs[p][1], [i16 + 16, l])
                        return 0

                    lax.fori_loop(rsv[0], rsv[1], ex, 0)

                    @pl.when(f + DEPTH < n_run)
                    def _nxt():
                        fpair(f + DEPTH, phs[p], sems[p], True)
            return 0

        lax.fori_loop(0, (n_run + DEPTH - 1) // DEPTH, outer, 0)

        cps = []
        for c in range(NCHK):
            cps.append(pltpu.async_copy(
                rows.at[pl.ds(c * CHK, CHK)], out_hbm.at[pv.at[c]], ssem))
        for cp in cps:
            cp.wait()

    one_pass(ut_g, ut_m, aug, aum, us_hbm, up_hbm, urows_hbm)
    one_pass(it_g, it_m, aig, aim, is_hbm, ip_hbm, irows_hbm)


_ROWS = jax.ShapeDtypeStruct((B, ROWW), jnp.float32)
_BLKT = pltpu.VMEM((D, LANE), jnp.float32)

_sc_gather = functools.partial(
    pl.kernel,
    out_type=(_ROWS, _ROWS),
    mesh=plsc.VectorSubcoreMesh(
        core_axis_name="c", subcore_axis_name="s",
        num_cores=NC, num_subcores=NS),
    scratch_types=(
        pltpu.VMEM((BPW + 32,), jnp.int32),
        pltpu.VMEM((NCHK, CHK), jnp.int32),
        pltpu.VMEM((BPW + 32,), jnp.int32),
        pltpu.VMEM((BPW, ROWW), jnp.float32),
        _BLKT, _BLKT, _BLKT, _BLKT, _BLKT, _BLKT, _BLKT, _BLKT,
        _BLKT, _BLKT, _BLKT, _BLKT,
        pltpu.SemaphoreType.DMA,
        pltpu.SemaphoreType.DMA,
        pltpu.SemaphoreType.DMA,
        pltpu.SemaphoreType.DMA,
        pltpu.SemaphoreType.DMA,
        pltpu.SemaphoreType.DMA,
        pltpu.SemaphoreType.DMA,
    ),
    compiler_params=pltpu.CompilerParams(needs_layout_passes=False),
)(_sc_gather_body)


def _tc_body(ur_ref, ir_ref, w1_ref, w2_ref, b1_ref,
             pg_ref, pm_ref, pb_ref, out_ref):
    ur = ur_ref[...]
    ir = ir_ref[...]
    gu, mu = ur[:, :D], ur[:, D:2 * D]
    gi, mi = ir[:, :D], ir[:, D:2 * D]
    h = jnp.dot(mu, w1_ref[...], preferred_element_type=jnp.float32)
    h = h + jnp.dot(mi, w2_ref[...], preferred_element_type=jnp.float32)
    h = jnp.maximum(h + b1_ref[...], 0.0)
    gmf = gu * gi
    out_ref[...] = (jnp.dot(gmf, pg_ref[...],
                            preferred_element_type=jnp.float32)
                    + jnp.dot(h, pm_ref[...],
                              preferred_element_type=jnp.float32)
                    + pb_ref[...])


_BLK = 2048


def kernel(user_indices, item_indices, gmf_user_emb, gmf_item_emb,
           mlp_user_emb, mlp_item_emb, fc1_w, fc1_b, pred_w, pred_b):
    uidx = user_indices.astype(jnp.int32)
    iidx = item_indices.astype(jnp.int32)
    keys = jnp.stack([uidx, iidx])
    pos = jnp.broadcast_to(jnp.arange(B, dtype=jnp.int32), (2, B))
    skeys, sperm = lax.sort((keys, pos), dimension=1, num_keys=1)
    us, isrt = skeys[0], skeys[1]
    up, ip = sperm[0], sperm[1]

    def _aux(tab):
        return jnp.pad(tab.T[:, LAST:], ((0, 0), (0, LANE - TAIL)))

    urows, irows = _sc_gather(
        gmf_user_emb.T, mlp_user_emb.T, gmf_item_emb.T, mlp_item_emb.T,
        _aux(gmf_user_emb), _aux(mlp_user_emb),
        _aux(gmf_item_emb), _aux(mlp_item_emb),
        us, up.reshape(B // CHK, CHK), isrt, ip.reshape(B // CHK, CHK))

    w1 = fc1_w[:, :D].T
    w2 = fc1_w[:, D:].T
    b1 = fc1_b.reshape(1, D)
    pg = pred_w[:, :D].T
    pm = pred_w[:, D:].T
    pb = pred_b.reshape(1, 1)

    rows_spec = pl.BlockSpec((_BLK, ROWW), lambda i: (i, 0))
    w_spec = pl.BlockSpec((D, D), lambda i: (0, 0))
    r_spec = pl.BlockSpec((1, D), lambda i: (0, 0))
    c_spec = pl.BlockSpec((D, 1), lambda i: (0, 0))
    s_spec = pl.BlockSpec((1, 1), lambda i: (0, 0))
    out2d = pl.pallas_call(
        _tc_body,
        grid=(B // _BLK,),
        in_specs=[rows_spec, rows_spec,
                  w_spec, w_spec, r_spec, c_spec, c_spec, s_spec],
        out_specs=pl.BlockSpec((_BLK, 1), lambda i: (i, 0)),
        out_shape=jax.ShapeDtypeStruct((B, 1), jnp.float32),
    )(urows, irows, w1, w2, b1, pg, pm, pb)
    return out2d.reshape(B)
